# baseline (speedup 1.0000x reference)
import functools

import numpy as np
import jax
import jax.numpy as jnp
from jax.experimental import pallas as pl
from jax.experimental.pallas import tpu as pltpu

_SCALE = 2


def _keys_cubic(t):
    t = np.abs(np.asarray(t, np.float64))
    return np.where(
        t <= 1.0,
        (1.5 * t - 2.5) * t * t + 1.0,
        np.where(t < 2.0, ((-0.5 * t + 2.5) * t - 4.0) * t + 2.0, 0.0),
    )


def _tap_weights(scale):
    offs = [(scale - 1) / (2.0 * scale) - j / float(scale) for j in range(scale)]
    wk = np.stack([_keys_cubic([b - 2, b - 1, b, b + 1, b + 2]) for b in offs])
    wk = wk / wk.sum(axis=1, keepdims=True)
    return wk.astype(np.float32)


def _h_matrix(w, scale):
    wk = _tap_weights(scale)
    m = np.zeros((w, w * scale), np.float32)
    cols = np.arange(w)
    for q in range(5):
        src = np.clip(cols + q - 2, 0, w - 1)
        for j in range(scale):
            np.add.at(m, (src, cols * scale + j), wk[j, q])
    return m


def _combined_tables(w):
    wk = _tap_weights(_SCALE)
    mh = _h_matrix(w, _SCALE)
    eye = np.eye(w, dtype=np.float32)
    tables = []
    for d in (-1, 0, 1):
        b = np.zeros((2 * w, 4 * w), np.float32)
        for t in range(2):
            for j in range(4):
                p = 2 * d + t + 2 - j // 2
                if 0 <= p <= 4:
                    b[t * w : (t + 1) * w, j * w : (j + 1) * w] = eye * wk[j % 2, p]
        mh_bd = np.zeros((4 * w, 8 * w), np.float32)
        for j in range(4):
            mh_bd[j * w : (j + 1) * w, 2 * j * w : 2 * (j + 1) * w] = mh
        tables.append(b @ mh_bd)
    return tables


def _upsample2x_kernel(x_ref, c_ref, o_ref):
    tile, hp, wp = x_ref.shape
    w = wp // 2
    xp = x_ref[...]

    left0 = xp[:, :1, :w]
    rightl = xp[:, hp - 1 :, w:]
    fix0 = jnp.concatenate([left0, left0], axis=2)
    fixl = jnp.concatenate([rightl, rightl], axis=2)

    xm = jnp.concatenate([fix0, xp[:, : hp - 1, :]], axis=1)
    xl = jnp.concatenate([xp[:, 1:, :], fixl], axis=1)

    a = jnp.concatenate([xm, xp, xl], axis=2).astype(jnp.bfloat16)
    m = tile * hp
    y = jnp.dot(a.reshape(m, 3 * wp), c_ref[...],
                preferred_element_type=jnp.float32)
    o_ref[...] = y.reshape(tile, 4 * hp, wp)


def kernel(x):
    b, c, h, w = x.shape
    scale = _SCALE
    hs, ws = h * scale, w * scale
    bc = b * c
    hp, wp = h // 2, 2 * w

    xr = x.reshape(bc, hp, wp)
    ctab = jnp.asarray(np.concatenate(_combined_tables(w), axis=0)).astype(
        jnp.bfloat16
    )

    tile = 64
    while bc % tile:
        tile //= 2
    grid = (bc // tile,)

    out = pl.pallas_call(
        _upsample2x_kernel,
        out_shape=jax.ShapeDtypeStruct((bc, 4 * hp, wp), x.dtype),
        grid=grid,
        in_specs=[
            pl.BlockSpec((tile, hp, wp), lambda i: (i, 0, 0)),
            pl.BlockSpec((3 * wp, 4 * wp), lambda i: (0, 0)),
        ],
        out_specs=pl.BlockSpec((tile, 4 * hp, wp), lambda i: (i, 0, 0)),
        compiler_params=pltpu.CompilerParams(
            dimension_semantics=("parallel",),
            vmem_limit_bytes=100 * 1024 * 1024,
        ),
    )(xr, ctab)

    return out.reshape(b, c, hs, ws)


# speedup vs baseline: 1.0558x; 1.0558x over previous
import functools

import numpy as np
import jax
import jax.numpy as jnp
from jax.experimental import pallas as pl
from jax.experimental.pallas import tpu as pltpu

_SCALE = 2


def _keys_cubic(t):
    t = np.abs(np.asarray(t, np.float64))
    return np.where(
        t <= 1.0,
        (1.5 * t - 2.5) * t * t + 1.0,
        np.where(t < 2.0, ((-0.5 * t + 2.5) * t - 4.0) * t + 2.0, 0.0),
    )


def _tap_weights(scale):
    offs = [(scale - 1) / (2.0 * scale) - j / float(scale) for j in range(scale)]
    wk = np.stack([_keys_cubic([b - 2, b - 1, b, b + 1, b + 2]) for b in offs])
    wk = wk / wk.sum(axis=1, keepdims=True)
    return wk.astype(np.float32)


def _h_matrix(w, scale):
    wk = _tap_weights(scale)
    m = np.zeros((w, w * scale), np.float32)
    cols = np.arange(w)
    for q in range(5):
        src = np.clip(cols + q - 2, 0, w - 1)
        for j in range(scale):
            np.add.at(m, (src, cols * scale + j), wk[j, q])
    return m


def _combined_tables(w):
    wk = _tap_weights(_SCALE)
    mh = _h_matrix(w, _SCALE)
    eye = np.eye(w, dtype=np.float32)
    tables = []
    for d in (-1, 0, 1):
        b = np.zeros((2 * w, 4 * w), np.float32)
        for t in range(2):
            for j in range(4):
                p = 2 * d + t + 2 - j // 2
                if 0 <= p <= 4:
                    b[t * w : (t + 1) * w, j * w : (j + 1) * w] = eye * wk[j % 2, p]
        mh_bd = np.zeros((4 * w, 8 * w), np.float32)
        for j in range(4):
            mh_bd[j * w : (j + 1) * w, 2 * j * w : 2 * (j + 1) * w] = mh
        tables.append(b @ mh_bd)
    return tables


def _upsample2x_kernel(x_ref, c_ref, o_ref):
    tile, hp, wp = x_ref.shape
    w = wp // 2
    xp = x_ref[...]

    left0 = xp[:, :1, :w]
    rightl = xp[:, hp - 1 :, w:]
    fix0 = jnp.concatenate([left0, left0], axis=2)
    fixl = jnp.concatenate([rightl, rightl], axis=2)

    xm = jnp.concatenate([fix0, xp[:, : hp - 1, :]], axis=1)
    xl = jnp.concatenate([xp[:, 1:, :], fixl], axis=1)

    a = jnp.concatenate([xm, xp, xl], axis=2).astype(jnp.bfloat16)
    m = tile * hp
    y = jnp.dot(a.reshape(m, 3 * wp), c_ref[...],
                preferred_element_type=jnp.float32)
    o_ref[...] = y.reshape(tile, 4 * hp, wp)


def kernel(x):
    b, c, h, w = x.shape
    scale = _SCALE
    hs, ws = h * scale, w * scale
    bc = b * c
    hp, wp = h // 2, 2 * w

    xr = x.reshape(bc, hp, wp)
    ctab = jnp.asarray(np.concatenate(_combined_tables(w), axis=0)).astype(
        jnp.bfloat16
    )

    tile = 256
    while bc % tile:
        tile //= 2
    grid = (bc // tile,)

    out = pl.pallas_call(
        _upsample2x_kernel,
        out_shape=jax.ShapeDtypeStruct((bc, 4 * hp, wp), x.dtype),
        grid=grid,
        in_specs=[
            pl.BlockSpec((tile, hp, wp), lambda i: (i, 0, 0)),
            pl.BlockSpec((3 * wp, 4 * wp), lambda i: (0, 0)),
        ],
        out_specs=pl.BlockSpec((tile, 4 * hp, wp), lambda i: (i, 0, 0)),
        compiler_params=pltpu.CompilerParams(
            dimension_semantics=("parallel",),
            vmem_limit_bytes=100 * 1024 * 1024,
        ),
    )(xr, ctab)

    return out.reshape(b, c, hs, ws)


# speedup vs baseline: 1.0657x; 1.0094x over previous
import numpy as np
import jax
import jax.numpy as jnp
from jax.experimental import pallas as pl
from jax.experimental.pallas import tpu as pltpu

_SCALE = 2


def _keys_cubic(t):
    t = np.abs(np.asarray(t, np.float64))
    return np.where(
        t <= 1.0,
        (1.5 * t - 2.5) * t * t + 1.0,
        np.where(t < 2.0, ((-0.5 * t + 2.5) * t - 4.0) * t + 2.0, 0.0),
    )


def _tap_weights(scale):
    offs = [(scale - 1) / (2.0 * scale) - j / float(scale) for j in range(scale)]
    wk = np.stack([_keys_cubic([b - 2, b - 1, b, b + 1, b + 2]) for b in offs])
    wk = wk / wk.sum(axis=1, keepdims=True)
    return wk.astype(np.float32)


def _h_matrix(w, scale):
    wk = _tap_weights(scale)
    m = np.zeros((w, w * scale), np.float32)
    cols = np.arange(w)
    for q in range(5):
        src = np.clip(cols + q - 2, 0, w - 1)
        for j in range(scale):
            np.add.at(m, (src, cols * scale + j), wk[j, q])
    return m


def _combined_tables(w):
    wk = _tap_weights(_SCALE)
    mh = _h_matrix(w, _SCALE)
    eye = np.eye(w, dtype=np.float32)
    tables = []
    for d in (-1, 0, 1):
        b = np.zeros((2 * w, 4 * w), np.float32)
        for t in range(2):
            for j in range(4):
                p = 2 * d + t + 2 - j // 2
                if 0 <= p <= 4:
                    b[t * w : (t + 1) * w, j * w : (j + 1) * w] = eye * wk[j % 2, p]
        mh_bd = np.zeros((4 * w, 8 * w), np.float32)
        for j in range(4):
            mh_bd[j * w : (j + 1) * w, 2 * j * w : 2 * (j + 1) * w] = mh
        tables.append(b @ mh_bd)
    return tables


def _upsample2x_kernel(x_ref, c_ref, o_ref):
    tile, hp, wp = x_ref.shape
    w = wp // 2
    xp = x_ref[...]

    left0 = xp[:, :1, :w]
    rightl = xp[:, hp - 1 :, w:]
    fix0 = jnp.concatenate([left0, left0], axis=2)
    fixl = jnp.concatenate([rightl, rightl], axis=2)

    xm = jnp.concatenate([fix0, xp[:, : hp - 1, :]], axis=1)
    xl = jnp.concatenate([xp[:, 1:, :], fixl], axis=1)

    a = jnp.concatenate([xm, xp, xl], axis=2).astype(jnp.bfloat16)
    m = tile * hp
    y = jnp.dot(a.reshape(m, 3 * wp), c_ref[...],
                preferred_element_type=jnp.float32)
    o_ref[...] = y.reshape(tile, 4 * hp, wp)


def kernel(x):
    b, c, h, w = x.shape
    scale = _SCALE
    hs, ws = h * scale, w * scale
    bc = b * c
    hp, wp = h // 2, 2 * w

    xr = x.reshape(bc, hp, wp)
    ctab = jnp.asarray(np.concatenate(_combined_tables(w), axis=0)).astype(
        jnp.bfloat16
    )

    tile = 128
    while bc % tile:
        tile //= 2
    grid = (bc // tile,)

    out = pl.pallas_call(
        _upsample2x_kernel,
        out_shape=jax.ShapeDtypeStruct((bc, 4 * hp, wp), x.dtype),
        grid=grid,
        in_specs=[
            pl.BlockSpec((tile, hp, wp), lambda i: (i, 0, 0)),
            pl.BlockSpec((3 * wp, 4 * wp), lambda i: (0, 0)),
        ],
        out_specs=pl.BlockSpec((tile, 4 * hp, wp), lambda i: (i, 0, 0)),
        compiler_params=pltpu.CompilerParams(
            dimension_semantics=("parallel",),
            vmem_limit_bytes=100 * 1024 * 1024,
        ),
    )(xr, ctab)

    return out.reshape(b, c, hs, ws)


# speedup vs baseline: 1.0673x; 1.0015x over previous
import numpy as np
import jax
import jax.numpy as jnp
from jax.experimental import pallas as pl
from jax.experimental.pallas import tpu as pltpu

_SCALE = 2


def _keys_cubic(t):
    t = np.abs(np.asarray(t, np.float64))
    return np.where(
        t <= 1.0,
        (1.5 * t - 2.5) * t * t + 1.0,
        np.where(t < 2.0, ((-0.5 * t + 2.5) * t - 4.0) * t + 2.0, 0.0),
    )


def _tap_weights(scale):
    offs = [(scale - 1) / (2.0 * scale) - j / float(scale) for j in range(scale)]
    wk = np.stack([_keys_cubic([b - 2, b - 1, b, b + 1, b + 2]) for b in offs])
    wk = wk / wk.sum(axis=1, keepdims=True)
    return wk.astype(np.float32)


def _h_matrix(w, scale):
    wk = _tap_weights(scale)
    m = np.zeros((w, w * scale), np.float32)
    cols = np.arange(w)
    for q in range(5):
        src = np.clip(cols + q - 2, 0, w - 1)
        for j in range(scale):
            np.add.at(m, (src, cols * scale + j), wk[j, q])
    return m


def _combined_tables(w):
    wk = _tap_weights(_SCALE)
    mh = _h_matrix(w, _SCALE)
    eye = np.eye(w, dtype=np.float32)
    tables = []
    for d in (-1, 0, 1):
        b = np.zeros((2 * w, 4 * w), np.float32)
        for t in range(2):
            for j in range(4):
                p = 2 * d + t + 2 - j // 2
                if 0 <= p <= 4:
                    b[t * w : (t + 1) * w, j * w : (j + 1) * w] = eye * wk[j % 2, p]
        mh_bd = np.zeros((4 * w, 8 * w), np.float32)
        for j in range(4):
            mh_bd[j * w : (j + 1) * w, 2 * j * w : 2 * (j + 1) * w] = mh
        tables.append(b @ mh_bd)
    return tables


def _upsample2x_kernel(x_ref, c_ref, o_ref):
    tile, hp, wp = x_ref.shape
    w = wp // 2
    xp = x_ref[...]

    left0 = xp[:, :1, :w]
    rightl = xp[:, hp - 1 :, w:]
    fix0 = jnp.concatenate([left0, left0], axis=2)
    fixl = jnp.concatenate([rightl, rightl], axis=2)

    xm = jnp.concatenate([fix0, xp[:, : hp - 1, :]], axis=1)
    xl = jnp.concatenate([xp[:, 1:, :], fixl], axis=1)

    a = jnp.concatenate([xm, xp, xl], axis=2).astype(jnp.bfloat16)
    ct = c_ref[...]
    half = tile // 2
    for i in range(2):
        ah = a[i * half : (i + 1) * half].reshape(half * hp, 3 * wp)
        y = jnp.dot(ah, ct, preferred_element_type=jnp.float32)
        o_ref[i * half : (i + 1) * half] = y.reshape(half, 4 * hp, wp)


def kernel(x):
    b, c, h, w = x.shape
    scale = _SCALE
    hs, ws = h * scale, w * scale
    bc = b * c
    hp, wp = h // 2, 2 * w

    xr = x.reshape(bc, hp, wp)
    ctab = jnp.asarray(np.concatenate(_combined_tables(w), axis=0)).astype(
        jnp.bfloat16
    )

    tile = 128
    while bc % tile:
        tile //= 2
    grid = (bc // tile,)

    out = pl.pallas_call(
        _upsample2x_kernel,
        out_shape=jax.ShapeDtypeStruct((bc, 4 * hp, wp), x.dtype),
        grid=grid,
        in_specs=[
            pl.BlockSpec((tile, hp, wp), lambda i: (i, 0, 0)),
            pl.BlockSpec((3 * wp, 4 * wp), lambda i: (0, 0)),
        ],
        out_specs=pl.BlockSpec((tile, 4 * hp, wp), lambda i: (i, 0, 0)),
        compiler_params=pltpu.CompilerParams(
            dimension_semantics=("parallel",),
            vmem_limit_bytes=100 * 1024 * 1024,
        ),
    )(xr, ctab)

    return out.reshape(b, c, hs, ws)


# speedup vs baseline: 1.0729x; 1.0052x over previous
import numpy as np
import jax
import jax.numpy as jnp
from jax.experimental import pallas as pl
from jax.experimental.pallas import tpu as pltpu

_SCALE = 2


def _keys_cubic(t):
    t = np.abs(np.asarray(t, np.float64))
    return np.where(
        t <= 1.0,
        (1.5 * t - 2.5) * t * t + 1.0,
        np.where(t < 2.0, ((-0.5 * t + 2.5) * t - 4.0) * t + 2.0, 0.0),
    )


def _tap_weights(scale):
    offs = [(scale - 1) / (2.0 * scale) - j / float(scale) for j in range(scale)]
    wk = np.stack([_keys_cubic([b - 2, b - 1, b, b + 1, b + 2]) for b in offs])
    wk = wk / wk.sum(axis=1, keepdims=True)
    return wk.astype(np.float32)


def _h_matrix(w, scale):
    wk = _tap_weights(scale)
    m = np.zeros((w, w * scale), np.float32)
    cols = np.arange(w)
    for q in range(5):
        src = np.clip(cols + q - 2, 0, w - 1)
        for j in range(scale):
            np.add.at(m, (src, cols * scale + j), wk[j, q])
    return m


def _combined_tables(w):
    wk = _tap_weights(_SCALE)
    mh = _h_matrix(w, _SCALE)
    eye = np.eye(w, dtype=np.float32)
    tables = []
    for d in (-1, 0, 1):
        b = np.zeros((2 * w, 4 * w), np.float32)
        for t in range(2):
            for j in range(4):
                p = 2 * d + t + 2 - j // 2
                if 0 <= p <= 4:
                    b[t * w : (t + 1) * w, j * w : (j + 1) * w] = eye * wk[j % 2, p]
        mh_bd = np.zeros((4 * w, 8 * w), np.float32)
        for j in range(4):
            mh_bd[j * w : (j + 1) * w, 2 * j * w : 2 * (j + 1) * w] = mh
        tables.append(b @ mh_bd)
    return tables


def _upsample2x_kernel(x_ref, c_ref, o_ref):
    tile, hp, wp = x_ref.shape
    w = wp // 2
    xp = x_ref[...]

    left0 = xp[:, :1, :w]
    rightl = xp[:, hp - 1 :, w:]
    fix0 = jnp.concatenate([left0, left0], axis=2)
    fixl = jnp.concatenate([rightl, rightl], axis=2)

    xm = jnp.concatenate([fix0, xp[:, : hp - 1, :]], axis=1)
    xl = jnp.concatenate([xp[:, 1:, :], fixl], axis=1)

    a = jnp.concatenate([xm, xp, xl], axis=2).astype(jnp.bfloat16)
    ct = c_ref[...]
    chunks = 2 if tile % 2 == 0 else 1
    half = tile // chunks
    for i in range(chunks):
        ah = a[i * half : (i + 1) * half].reshape(half * hp, 3 * wp)
        y = jnp.dot(ah, ct, preferred_element_type=jnp.float32)
        o_ref[i * half : (i + 1) * half] = y.reshape(half, 4 * hp, wp)


def kernel(x):
    b, c, h, w = x.shape
    scale = _SCALE
    hs, ws = h * scale, w * scale
    bc = b * c
    hp, wp = h // 2, 2 * w

    xr = x.reshape(bc, hp, wp)
    ctab = jnp.asarray(np.concatenate(_combined_tables(w), axis=0)).astype(
        jnp.bfloat16
    )

    tile = 128
    while bc % tile:
        tile //= 2
    grid = (bc // tile,)

    out = pl.pallas_call(
        _upsample2x_kernel,
        out_shape=jax.ShapeDtypeStruct((bc, 4 * hp, wp), x.dtype),
        grid=grid,
        in_specs=[
            pl.BlockSpec((tile, hp, wp), lambda i: (i, 0, 0)),
            pl.BlockSpec((3 * wp, 4 * wp), lambda i: (0, 0)),
        ],
        out_specs=pl.BlockSpec((tile, 4 * hp, wp), lambda i: (i, 0, 0)),
        compiler_params=pltpu.CompilerParams(
            dimension_semantics=("parallel",),
            vmem_limit_bytes=100 * 1024 * 1024,
        ),
    )(xr, ctab)

    return out.reshape(b, c, hs, ws)
